# Initial kernel scaffold; baseline (speedup 1.0000x reference)
import jax
import jax.numpy as jnp
from jax.experimental import pallas as pl


def kernel(x, edge_index, W, b):
    raise NotImplementedError("write your pallas kernel here")


# speedup vs baseline: 12.8365x; 12.8365x over previous
import functools

import jax
import jax.numpy as jnp
from jax import lax
from jax.experimental import pallas as pl
from jax.experimental.pallas import tpu as pltpu
from jax.experimental.pallas import tpu_sc as plsc

N_NODES = 10000
DIM_IN = 256
DIM_OUT = 256
N_EDGES = 160000

NC = 2
NS = 16
NW = NC * NS

CH = 80
EPAD = ((N_EDGES + NW * CH - 1) // (NW * CH)) * (NW * CH)
K1C = EPAD // (NW * CH)
K3C = EPAD // (NS * CH)
NPAD = 10240
SLC = NPAD // NS
DH = DIM_OUT // NC
NB = 10
BN = N_NODES // NB

_mesh = plsc.VectorSubcoreMesh(core_axis_name="c", subcore_axis_name="s")


@functools.partial(
    pl.kernel,
    mesh=_mesh,
    out_type=jax.ShapeDtypeStruct((NC, NS, SLC), jnp.float32),
    scratch_types=[
        pltpu.VMEM((CH,), jnp.float32),
        pltpu.VMEM((K1C, CH), jnp.int32),
        pltpu.VMEM_SHARED((NPAD,), jnp.float32),
    ],
)
def _deg_kernel(dst_hbm, zeros1_hbm, ones_hbm, deg_out, ones_v, dsti_v, deg_acc):
    c = lax.axis_index("c")
    s = lax.axis_index("s")
    wid = s * NC + c
    pltpu.sync_copy(zeros1_hbm, deg_acc.at[pl.ds(s * SLC, SLC)])
    pltpu.sync_copy(ones_hbm, ones_v)
    pltpu.sync_copy(dst_hbm.at[wid], dsti_v)
    plsc.subcore_barrier()

    def body(j, carry):
        pltpu.sync_copy(ones_v, deg_acc.at[dsti_v.at[j]], add=True)
        return carry

    lax.fori_loop(0, K1C, body, 0)
    plsc.subcore_barrier()
    pltpu.sync_copy(deg_acc.at[pl.ds(s * SLC, SLC)], deg_out.at[c].at[s])


@functools.partial(
    pl.kernel,
    mesh=_mesh,
    out_type=jax.ShapeDtypeStruct((NC, NPAD, DH), jnp.float32),
    scratch_types=[
        pltpu.VMEM((K3C, CH), jnp.int32),
        pltpu.VMEM((K3C, CH), jnp.int32),
        pltpu.VMEM((CH, DH), jnp.float32),
        pltpu.VMEM_SHARED((NPAD, DH), jnp.float32),
        pltpu.SemaphoreType.DMA,
    ],
)
def _agg_kernel(hs_hbm, srcs_hbm, dst_hbm, zeros2_hbm, agg_out,
                srci_v, dsti_v, rows_v, acc, sem):
    c = lax.axis_index("c")
    s = lax.axis_index("s")
    pltpu.sync_copy(zeros2_hbm, acc.at[pl.ds(s * SLC, SLC)])
    pltpu.sync_copy(srcs_hbm.at[c].at[s], srci_v)
    pltpu.sync_copy(dst_hbm.at[s], dsti_v)
    plsc.subcore_barrier()

    def body(j, carry):
        pltpu.async_copy(hs_hbm.at[srci_v.at[j]], rows_v, sem).wait()
        pltpu.sync_copy(rows_v, acc.at[dsti_v.at[j]], add=True)
        return carry

    lax.fori_loop(0, K3C, body, 0)
    plsc.subcore_barrier()
    pltpu.sync_copy(acc.at[pl.ds(s * SLC, SLC)],
                    agg_out.at[c].at[pl.ds(s * SLC, SLC)])


def _mm_body(x_ref, w_ref, deg_ref, hs_ref, dis_ref):
    h = jnp.dot(x_ref[...], w_ref[...], preferred_element_type=jnp.float32)
    deg = deg_ref[:, 0] + deg_ref[:, 1] + 1.0
    dis = lax.rsqrt(deg)
    hs = h * dis[:, None]
    hs_ref[0] = hs[:, :DH]
    hs_ref[1] = hs[:, DH:]
    dis_ref[...] = dis[:, None]


def _mm(x, W, deg2):
    return pl.pallas_call(
        _mm_body,
        grid=(NB,),
        in_specs=[
            pl.BlockSpec((BN, DIM_IN), lambda i: (i, 0)),
            pl.BlockSpec((DIM_IN, DIM_OUT), lambda i: (0, 0)),
            pl.BlockSpec((BN, NC), lambda i: (i, 0)),
        ],
        out_specs=[
            pl.BlockSpec((NC, BN, DH), lambda i: (0, i, 0)),
            pl.BlockSpec((BN, 1), lambda i: (i, 0)),
        ],
        out_shape=[
            jax.ShapeDtypeStruct((NC, N_NODES, DH), jnp.float32),
            jax.ShapeDtypeStruct((N_NODES, 1), jnp.float32),
        ],
    )(x, W, deg2)


def _ep_body(agg_ref, hs_ref, dis_ref, b_ref, out_ref):
    a0 = agg_ref[0] + hs_ref[0]
    a1 = agg_ref[1] + hs_ref[1]
    full = jnp.concatenate([a0, a1], axis=1)
    out_ref[...] = full * dis_ref[...] + b_ref[...]


def _epilogue(agg, hs, dis, b2):
    return pl.pallas_call(
        _ep_body,
        grid=(NB,),
        in_specs=[
            pl.BlockSpec((NC, BN, DH), lambda i: (0, i, 0)),
            pl.BlockSpec((NC, BN, DH), lambda i: (0, i, 0)),
            pl.BlockSpec((BN, 1), lambda i: (i, 0)),
            pl.BlockSpec((1, DIM_OUT), lambda i: (0, 0)),
        ],
        out_specs=pl.BlockSpec((BN, DIM_OUT), lambda i: (i, 0)),
        out_shape=jax.ShapeDtypeStruct((N_NODES, DIM_OUT), jnp.float32),
    )(agg, hs, dis, b2)


def kernel(x, edge_index, W, b):
    ei = edge_index.astype(jnp.int32)
    src = ei[0]
    dst = ei[1]
    npadE = EPAD - N_EDGES
    src_p = jnp.concatenate([src, jnp.zeros((npadE,), jnp.int32)])
    dst_p = jnp.concatenate([dst, jnp.full((npadE,), N_NODES, jnp.int32)])
    dst_k1 = dst_p.reshape(NW, K1C, CH)
    dst_k3 = dst_p.reshape(NS, K3C, CH)
    srcs = jnp.stack([src_p, src_p + N_NODES]).reshape(NC, NS, K3C, CH)

    zeros1 = jnp.zeros((SLC,), jnp.float32)
    zeros2 = jnp.zeros((SLC, DH), jnp.float32)
    ones = jnp.ones((CH,), jnp.float32)

    deg_out = _deg_kernel(dst_k1, zeros1, ones)
    deg2 = deg_out.reshape(NC, NPAD)[:, :N_NODES].T

    hs, dis = _mm(x, W, deg2)

    hs_flat = hs.reshape(NC * N_NODES, DH)
    agg = _agg_kernel(hs_flat, srcs, dst_k3, zeros2)

    return _epilogue(agg, hs, dis, b.reshape(1, DIM_OUT))
